# baseline (speedup 1.0000x reference)
import functools

import jax
import jax.numpy as jnp
from jax import lax
from jax.experimental import pallas as pl
from jax.experimental.pallas import tpu as pltpu
from jax.experimental.pallas import tpu_sc as plsc

N = 10000
E = 320000
DN = 128
DE = 16
H = 128

NC = 2
NS = 16
L = 16
NW = NC * NS

_MESH = plsc.VectorSubcoreMesh(
    core_axis_name="c", subcore_axis_name="s", num_cores=NC, num_subcores=NS
)
_SC_PARAMS = pltpu.CompilerParams(use_tc_tiling_on_sc=False)
_SC_PARAMS_NL = pltpu.CompilerParams(
    use_tc_tiling_on_sc=False, needs_layout_passes=False
)

CH = 128
KA = 80
KD = 160
DEPTH = 4
NP = 10112
DNP = 20096
TRASH_A = NP - 1
TRASH_D = DNP - 1
DROWS = DNP // NS
AROWS = NP // NS
ZR = 8
EPT = E // NW


def _worker_id():
    return lax.axis_index("s") * NC + lax.axis_index("c")


def _deg_body(idx2_hbm, out_hbm, deg_sh, ones_v, idxw_v, zrow_v, ssem):
    cid = lax.axis_index("c")
    sid = lax.axis_index("s")
    wid = _worker_id()

    @pl.loop(0, CH)
    def _(i):
        ones_v[i, :] = jnp.full((L,), 1.0, jnp.float32)

    @pl.loop(0, DROWS)
    def _(i):
        zrow_v[i, :] = jnp.zeros((L,), jnp.float32)

    pltpu.sync_copy(zrow_v, deg_sh.at[pl.ds(sid * DROWS, DROWS)])
    pltpu.sync_copy(idx2_hbm.at[wid], idxw_v)
    plsc.subcore_barrier()

    @pl.loop(0, KD // DEPTH)
    def _(j):
        for b in range(DEPTH):
            k = j * DEPTH + b

            @pl.when(j >= 1)
            def _():
                pltpu.make_async_copy(
                    ones_v, deg_sh.at[idxw_v.at[0]], ssem.at[b]
                ).wait()

            pltpu.async_copy(ones_v, deg_sh.at[idxw_v.at[k]], ssem.at[b], add=True)

    for b in range(DEPTH):
        pltpu.make_async_copy(ones_v, deg_sh.at[idxw_v.at[0]], ssem.at[b]).wait()

    plsc.subcore_barrier()
    pltpu.sync_copy(
        deg_sh.at[pl.ds(sid * DROWS, DROWS)],
        out_hbm.at[cid, pl.ds(sid * DROWS, DROWS)],
    )


_deg_call = pl.kernel(
    _deg_body,
    out_type=jax.ShapeDtypeStruct((NC, DNP, L), jnp.float32),
    mesh=_MESH,
    scratch_types=[
        pltpu.VMEM_SHARED((DNP, L), jnp.float32),
        pltpu.VMEM((CH, L), jnp.float32),
        pltpu.VMEM((KD, CH), jnp.int32),
        pltpu.VMEM((DROWS, L), jnp.float32),
        pltpu.SemaphoreType.DMA((DEPTH,)),
    ],
    compiler_params=_SC_PARAMS,
)


AGG_CHUNKS = E // CH
AGG_KMAX = -(-AGG_CHUNKS // NW)


def _agg_body(hs_hbm, src_hbm, dst_hbm, out_hbm, agg_sh, rows_v, sidx_v,
              didx_v, zbuf_v, gsem, ssem, isem):
    cid = lax.axis_index("c")
    sid = lax.axis_index("s")
    wid = _worker_id()

    @pl.loop(0, ZR)
    def _(i):
        @pl.loop(0, H // L)
        def _(j):
            zbuf_v[i, pl.ds(j * L, L)] = jnp.zeros((L,), jnp.float32)

    @pl.loop(0, AROWS // ZR)
    def _(r):
        pltpu.sync_copy(zbuf_v, agg_sh.at[pl.ds(sid * AROWS + r * ZR, ZR)])

    for k0 in (0, 1):
        c0 = wid + k0 * NW

        @pl.when(c0 < AGG_CHUNKS)
        def _():
            pltpu.async_copy(src_hbm.at[pl.ds(c0 * CH, CH)], sidx_v.at[k0],
                             isem.at[k0])
            pltpu.async_copy(dst_hbm.at[pl.ds(c0 * CH, CH)], didx_v.at[k0],
                             isem.at[k0])

    plsc.subcore_barrier()

    @pl.loop(0, (AGG_KMAX + 4) // 4)
    def _(j):
        for u in range(4):
            k = 4 * j + u
            c = wid + k * NW
            half = u % 2
            rows_h = rows_v.at[half]
            sidx_u = sidx_v.at[u]
            didx_u = didx_v.at[u]

            @pl.when(c < AGG_CHUNKS)
            def _():
                @pl.when(k >= 2)
                def _():
                    pltpu.make_async_copy(
                        rows_h, agg_sh.at[didx_u], ssem.at[half]
                    ).wait()

                @pl.when(c + 2 * NW < AGG_CHUNKS)
                def _():
                    c2 = c + 2 * NW
                    u2 = (u + 2) % 4
                    pltpu.async_copy(src_hbm.at[pl.ds(c2 * CH, CH)],
                                     sidx_v.at[u2], isem.at[u2])
                    pltpu.async_copy(dst_hbm.at[pl.ds(c2 * CH, CH)],
                                     didx_v.at[u2], isem.at[u2])

                pltpu.make_async_copy(src_hbm.at[pl.ds(c * CH, CH)], sidx_u,
                                      isem.at[u]).wait()
                pltpu.make_async_copy(dst_hbm.at[pl.ds(c * CH, CH)], didx_u,
                                      isem.at[u]).wait()
                pltpu.async_copy(hs_hbm.at[sidx_u], rows_h, gsem.at[half]).wait()
                pltpu.async_copy(rows_h, agg_sh.at[didx_u], ssem.at[half],
                                 add=True)

    for half in (0, 1):
        pltpu.make_async_copy(
            rows_v.at[half], agg_sh.at[didx_v.at[half]], ssem.at[half]
        ).wait()

    plsc.subcore_barrier()
    pltpu.sync_copy(
        agg_sh.at[pl.ds(sid * AROWS, AROWS)],
        out_hbm.at[cid, pl.ds(sid * AROWS, AROWS)],
    )


_agg_call = pl.kernel(
    _agg_body,
    out_type=jax.ShapeDtypeStruct((NC, NP, H), jnp.float32),
    mesh=_MESH,
    scratch_types=[
        pltpu.VMEM_SHARED((NP, H), jnp.float32),
        pltpu.VMEM((2, CH, H), jnp.float32),
        pltpu.VMEM((4, CH), jnp.int32),
        pltpu.VMEM((4, CH), jnp.int32),
        pltpu.VMEM((ZR, H), jnp.float32),
        pltpu.SemaphoreType.DMA((2,)),
        pltpu.SemaphoreType.DMA((2,)),
        pltpu.SemaphoreType.DMA((4,)),
    ],
    compiler_params=_SC_PARAMS,
)


def _score_body(ps_hbm, pd_hbm, src_hbm, dst_hbm, pe_hbm, out_hbm,
                ps_v, pd_v, sidx_v, didx_v, pe_v, out_v):
    wid = _worker_id()
    base = wid * EPT
    pltpu.sync_copy(ps_hbm, ps_v)
    pltpu.sync_copy(pd_hbm, pd_v)
    pltpu.sync_copy(src_hbm.at[pl.ds(base, EPT)], sidx_v)
    pltpu.sync_copy(dst_hbm.at[pl.ds(base, EPT)], didx_v)
    pltpu.sync_copy(pe_hbm.at[pl.ds(base, EPT)], pe_v)

    @pl.loop(0, EPT // L)
    def _(i):
        si = sidx_v[pl.ds(i * L, L)]
        di = didx_v[pl.ds(i * L, L)]
        vs = plsc.load_gather(ps_v, [si])
        vd = plsc.load_gather(pd_v, [di])
        out_v[pl.ds(i * L, L)] = vs + vd + pe_v[pl.ds(i * L, L)]

    pltpu.sync_copy(out_v, out_hbm.at[pl.ds(base, EPT)])


_score_call = pl.kernel(
    _score_body,
    out_type=jax.ShapeDtypeStruct((E,), jnp.float32),
    mesh=_MESH,
    scratch_types=[
        pltpu.VMEM((N,), jnp.float32),
        pltpu.VMEM((N,), jnp.float32),
        pltpu.VMEM((EPT,), jnp.int32),
        pltpu.VMEM((EPT,), jnp.int32),
        pltpu.VMEM((EPT,), jnp.float32),
        pltpu.VMEM((EPT,), jnp.float32),
    ],
    compiler_params=_SC_PARAMS_NL,
)


_NBLK = 1000
_NGRID = N // _NBLK
_EBLK = 8000
_EGRID = E // _EBLK


def _node_embed_tc(x_ref, w_ref, b_ref, out_ref):
    out_ref[...] = (
        jnp.dot(x_ref[...], w_ref[...], preferred_element_type=jnp.float32)
        + b_ref[...]
    )


def _pe_tc(e_ref, wedge_ref, wpred_ref, bedge_ref, bpred_ref, out_ref):
    wpe = wpred_ref[...][2 * H:3 * H, :]
    wsmall = jnp.dot(wedge_ref[...], wpe, preferred_element_type=jnp.float32)
    c = jnp.dot(bedge_ref[...], wpe, preferred_element_type=jnp.float32)
    out_ref[...] = (
        jnp.dot(e_ref[...], wsmall, preferred_element_type=jnp.float32)
        + c + bpred_ref[...]
    )


def _norm_tc(dps_ref, dpd_ref, h0_ref, hs_ref, rso_ref, rsi_ref):
    ds_ = dps_ref[...]
    dd_ = dpd_ref[...]
    deg_s = ds_[0, :, 0:1] + ds_[1, :, 0:1]
    deg_d = dd_[0, :, 0:1] + dd_[1, :, 0:1]
    rso = lax.rsqrt(jnp.maximum(deg_s, 1.0))
    rsi = lax.rsqrt(jnp.maximum(deg_d, 1.0))
    hs_ref[...] = h0_ref[...] * rso
    rso_ref[...] = rso
    rsi_ref[...] = rsi


def _layer_tc(aggp_ref, rsi_ref, rso_ref, w_ref, b_ref, out_ref):
    a = aggp_ref[...]
    agg = (a[0] + a[1]) * rsi_ref[...]
    h = jnp.maximum(
        jnp.dot(agg, w_ref[...], preferred_element_type=jnp.float32) + b_ref[...],
        0.0,
    )
    out_ref[...] = h * rso_ref[...]


def _copy_tc(x_ref, out_ref):
    out_ref[...] = x_ref[...]


def _final_tc(aggp_ref, rsi_ref, w_ref, b_ref, wpred_ref, ps_ref, pd_ref):
    a = aggp_ref[...]
    agg = (a[0] + a[1]) * rsi_ref[...]
    h = jnp.maximum(
        jnp.dot(agg, w_ref[...], preferred_element_type=jnp.float32) + b_ref[...],
        0.0,
    )
    wp = wpred_ref[...]
    ps_ref[...] = jnp.dot(h, wp[0:H, :], preferred_element_type=jnp.float32)
    pd_ref[...] = jnp.dot(h, wp[H:2 * H, :], preferred_element_type=jnp.float32)


def _full(shape):
    return pl.BlockSpec(shape, lambda i: (0,) * len(shape))


_node_embed_call = pl.pallas_call(
    _node_embed_tc,
    grid=(_NGRID,),
    in_specs=[
        pl.BlockSpec((_NBLK, DN), lambda i: (i, 0)),
        _full((DN, H)),
        _full((1, H)),
    ],
    out_specs=pl.BlockSpec((_NBLK, H), lambda i: (i, 0)),
    out_shape=jax.ShapeDtypeStruct((N, H), jnp.float32),
)

_pe_call = pl.pallas_call(
    _pe_tc,
    grid=(_EGRID,),
    in_specs=[
        pl.BlockSpec((_EBLK, DE), lambda i: (i, 0)),
        _full((DE, H)),
        _full((3 * H, 1)),
        _full((1, H)),
        _full((1, 1)),
    ],
    out_specs=pl.BlockSpec((_EBLK, 1), lambda i: (i, 0)),
    out_shape=jax.ShapeDtypeStruct((E, 1), jnp.float32),
)

_norm_call = pl.pallas_call(
    _norm_tc,
    grid=(_NGRID,),
    in_specs=[
        pl.BlockSpec((NC, _NBLK, L), lambda i: (0, i, 0)),
        pl.BlockSpec((NC, _NBLK, L), lambda i: (0, i + _NGRID, 0)),
        pl.BlockSpec((_NBLK, H), lambda i: (i, 0)),
    ],
    out_specs=[
        pl.BlockSpec((_NBLK, H), lambda i: (i, 0)),
        pl.BlockSpec((_NBLK, 1), lambda i: (i, 0)),
        pl.BlockSpec((_NBLK, 1), lambda i: (i, 0)),
    ],
    out_shape=[
        jax.ShapeDtypeStruct((N, H), jnp.float32),
        jax.ShapeDtypeStruct((N, 1), jnp.float32),
        jax.ShapeDtypeStruct((N, 1), jnp.float32),
    ],
)

_layer_call = pl.pallas_call(
    _layer_tc,
    grid=(_NGRID,),
    in_specs=[
        pl.BlockSpec((NC, _NBLK, H), lambda i: (0, i, 0)),
        pl.BlockSpec((_NBLK, 1), lambda i: (i, 0)),
        pl.BlockSpec((_NBLK, 1), lambda i: (i, 0)),
        _full((H, H)),
        _full((1, H)),
    ],
    out_specs=pl.BlockSpec((_NBLK, H), lambda i: (i, 0)),
    out_shape=jax.ShapeDtypeStruct((N, H), jnp.float32),
)

_copy_call = pl.pallas_call(
    _copy_tc,
    grid=(_NGRID,),
    in_specs=[pl.BlockSpec((_NBLK, H), lambda i: (i, 0))],
    out_specs=pl.BlockSpec((_NBLK, H), lambda i: (i, 0)),
    out_shape=jax.ShapeDtypeStruct((N, H), jnp.float32),
)

_final_call = pl.pallas_call(
    _final_tc,
    grid=(_NGRID,),
    in_specs=[
        pl.BlockSpec((NC, _NBLK, H), lambda i: (0, i, 0)),
        pl.BlockSpec((_NBLK, 1), lambda i: (i, 0)),
        _full((H, H)),
        _full((1, H)),
        _full((3 * H, 1)),
    ],
    out_specs=[
        pl.BlockSpec((_NBLK, 1), lambda i: (i, 0)),
        pl.BlockSpec((_NBLK, 1), lambda i: (i, 0)),
    ],
    out_shape=[
        jax.ShapeDtypeStruct((N, 1), jnp.float32),
        jax.ShapeDtypeStruct((N, 1), jnp.float32),
    ],
)


def kernel(x, e, edge_index, W_node, b_node, W_edge, b_edge, Wg0, bg0, Wg1, bg1,
           W_pred, b_pred):
    src = edge_index[0].astype(jnp.int32)
    dst = edge_index[1].astype(jnp.int32)
    idx2 = jnp.concatenate([src, dst + N])

    npad_d = KD * CH - 2 * E // NW
    pad_d = (2 * N + jnp.arange(NW * npad_d, dtype=jnp.int32) % (DNP - 2 * N)
             ).reshape(NW, npad_d)
    idx2p = jnp.concatenate(
        [idx2.reshape(NW, 2 * E // NW), pad_d], axis=1
    ).reshape(NW, KD, CH)

    h0 = _node_embed_call(x, W_node, b_node.reshape(1, H))
    pe = _pe_call(e, W_edge, W_pred, b_edge.reshape(1, H), b_pred.reshape(1, 1))
    degp = _deg_call(idx2p)

    hs1, rso, rsi = _norm_call(degp, degp, h0)
    aggp1 = _agg_call(hs1, src, dst)
    hs2 = _copy_call(_layer_call(aggp1, rsi, rso, Wg0, bg0.reshape(1, H)))
    aggp2 = _agg_call(hs2, src, dst)
    ps, pd = _final_call(aggp2, rsi, Wg1, bg1.reshape(1, H), W_pred)

    scores = _score_call(ps.reshape(N), pd.reshape(N), src, dst, pe.reshape(E))
    return scores.reshape(E, 1)


# speedup vs baseline: 1.2525x; 1.2525x over previous
import functools

import jax
import jax.numpy as jnp
from jax import lax
from jax.experimental import pallas as pl
from jax.experimental.pallas import tpu as pltpu
from jax.experimental.pallas import tpu_sc as plsc

N = 10000
E = 320000
DN = 128
DE = 16
H = 128

NC = 2
NS = 16
L = 16
NW = NC * NS

_MESH = plsc.VectorSubcoreMesh(
    core_axis_name="c", subcore_axis_name="s", num_cores=NC, num_subcores=NS
)
_SC_PARAMS = pltpu.CompilerParams(use_tc_tiling_on_sc=False)
_SC_PARAMS_NL = pltpu.CompilerParams(
    use_tc_tiling_on_sc=False, needs_layout_passes=False
)

CH = 128
KA = 80
KD = 160
DEPTH = 4
NP = 10112
DNP = 20096
TRASH_A = NP - 1
TRASH_D = DNP - 1
DROWS = DNP // NS
AROWS = NP // NS
ZR = 8
EPT = E // NW


def _worker_id():
    return lax.axis_index("s") * NC + lax.axis_index("c")


def _deg_body(idx2_hbm, out_hbm, deg_sh, ones_v, idxw_v, zrow_v, ssem):
    cid = lax.axis_index("c")
    sid = lax.axis_index("s")
    wid = _worker_id()

    @pl.loop(0, CH)
    def _(i):
        ones_v[i, :] = jnp.full((L,), 1.0, jnp.float32)

    @pl.loop(0, DROWS)
    def _(i):
        zrow_v[i, :] = jnp.zeros((L,), jnp.float32)

    pltpu.sync_copy(zrow_v, deg_sh.at[pl.ds(sid * DROWS, DROWS)])
    pltpu.sync_copy(idx2_hbm.at[wid], idxw_v)
    plsc.subcore_barrier()

    @pl.loop(0, KD // DEPTH)
    def _(j):
        for b in range(DEPTH):
            k = j * DEPTH + b

            @pl.when(j >= 1)
            def _():
                pltpu.make_async_copy(
                    ones_v, deg_sh.at[idxw_v.at[0]], ssem.at[b]
                ).wait()

            pltpu.async_copy(ones_v, deg_sh.at[idxw_v.at[k]], ssem.at[b], add=True)

    for b in range(DEPTH):
        pltpu.make_async_copy(ones_v, deg_sh.at[idxw_v.at[0]], ssem.at[b]).wait()

    plsc.subcore_barrier()
    pltpu.sync_copy(
        deg_sh.at[pl.ds(sid * DROWS, DROWS)],
        out_hbm.at[cid, pl.ds(sid * DROWS, DROWS)],
    )


_deg_call = pl.kernel(
    _deg_body,
    out_type=jax.ShapeDtypeStruct((NC, DNP, L), jnp.float32),
    mesh=_MESH,
    scratch_types=[
        pltpu.VMEM_SHARED((DNP, L), jnp.float32),
        pltpu.VMEM((CH, L), jnp.float32),
        pltpu.VMEM((KD, CH), jnp.int32),
        pltpu.VMEM((DROWS, L), jnp.float32),
        pltpu.SemaphoreType.DMA((DEPTH,)),
    ],
    compiler_params=_SC_PARAMS,
)


AGG_CHUNKS = E // CH
AGG_KMAX = -(-AGG_CHUNKS // NW)


def _agg_body(hs_hbm, src_hbm, dst_hbm, out_hbm, agg_sh, rows_v, sidx_v,
              didx_v, zbuf_v, gsem, ssem, isem):
    cid = lax.axis_index("c")
    sid = lax.axis_index("s")
    wid = _worker_id()

    @pl.loop(0, ZR)
    def _(i):
        @pl.loop(0, H // L)
        def _(j):
            zbuf_v[i, pl.ds(j * L, L)] = jnp.zeros((L,), jnp.float32)

    @pl.loop(0, AROWS // ZR)
    def _(r):
        pltpu.sync_copy(zbuf_v, agg_sh.at[pl.ds(sid * AROWS + r * ZR, ZR)])

    for k0 in (0, 1):
        c0 = wid + k0 * NW

        @pl.when(c0 < AGG_CHUNKS)
        def _():
            pltpu.async_copy(src_hbm.at[pl.ds(c0 * CH, CH)], sidx_v.at[k0],
                             isem.at[k0])
            pltpu.async_copy(dst_hbm.at[pl.ds(c0 * CH, CH)], didx_v.at[k0],
                             isem.at[k0])

    plsc.subcore_barrier()

    @pl.loop(0, (AGG_KMAX + 4) // 4)
    def _(j):
        for u in range(4):
            k = 4 * j + u
            c = wid + k * NW
            half = u % 2
            rows_h = rows_v.at[half]
            sidx_u = sidx_v.at[u]
            didx_u = didx_v.at[u]

            @pl.when(c < AGG_CHUNKS)
            def _():
                @pl.when(k >= 2)
                def _():
                    pltpu.make_async_copy(
                        rows_h, agg_sh.at[didx_u], ssem.at[half]
                    ).wait()

                @pl.when(c + 2 * NW < AGG_CHUNKS)
                def _():
                    c2 = c + 2 * NW
                    u2 = (u + 2) % 4
                    pltpu.async_copy(src_hbm.at[pl.ds(c2 * CH, CH)],
                                     sidx_v.at[u2], isem.at[u2])
                    pltpu.async_copy(dst_hbm.at[pl.ds(c2 * CH, CH)],
                                     didx_v.at[u2], isem.at[u2])

                pltpu.make_async_copy(src_hbm.at[pl.ds(c * CH, CH)], sidx_u,
                                      isem.at[u]).wait()
                pltpu.make_async_copy(dst_hbm.at[pl.ds(c * CH, CH)], didx_u,
                                      isem.at[u]).wait()
                pltpu.async_copy(hs_hbm.at[sidx_u], rows_h, gsem.at[half]).wait()
                pltpu.async_copy(rows_h, agg_sh.at[didx_u], ssem.at[half],
                                 add=True)

    for half in (0, 1):
        pltpu.make_async_copy(
            rows_v.at[half], agg_sh.at[didx_v.at[half]], ssem.at[half]
        ).wait()

    plsc.subcore_barrier()
    pltpu.sync_copy(
        agg_sh.at[pl.ds(sid * AROWS, AROWS)],
        out_hbm.at[cid, pl.ds(sid * AROWS, AROWS)],
    )


_agg_call = pl.kernel(
    _agg_body,
    out_type=jax.ShapeDtypeStruct((NC, NP, H), jnp.float32),
    mesh=_MESH,
    scratch_types=[
        pltpu.VMEM_SHARED((NP, H), jnp.float32),
        pltpu.VMEM((2, CH, H), jnp.float32),
        pltpu.VMEM((4, CH), jnp.int32),
        pltpu.VMEM((4, CH), jnp.int32),
        pltpu.VMEM((ZR, H), jnp.float32),
        pltpu.SemaphoreType.DMA((2,)),
        pltpu.SemaphoreType.DMA((2,)),
        pltpu.SemaphoreType.DMA((4,)),
    ],
    compiler_params=_SC_PARAMS,
)


def _score_body(ps_hbm, pd_hbm, src_hbm, dst_hbm, pe_hbm, out_hbm,
                ps_v, pd_v, sidx_v, didx_v, pe_v, out_v):
    wid = _worker_id()
    base = wid * EPT
    pltpu.sync_copy(ps_hbm, ps_v)
    pltpu.sync_copy(pd_hbm, pd_v)
    pltpu.sync_copy(src_hbm.at[pl.ds(base, EPT)], sidx_v)
    pltpu.sync_copy(dst_hbm.at[pl.ds(base, EPT)], didx_v)
    pltpu.sync_copy(pe_hbm.at[pl.ds(base, EPT)], pe_v)

    @pl.loop(0, EPT // L)
    def _(i):
        si = sidx_v[pl.ds(i * L, L)]
        di = didx_v[pl.ds(i * L, L)]
        vs = plsc.load_gather(ps_v, [si])
        vd = plsc.load_gather(pd_v, [di])
        out_v[pl.ds(i * L, L)] = vs + vd + pe_v[pl.ds(i * L, L)]

    pltpu.sync_copy(out_v, out_hbm.at[pl.ds(base, EPT)])


_score_call = pl.kernel(
    _score_body,
    out_type=jax.ShapeDtypeStruct((E,), jnp.float32),
    mesh=_MESH,
    scratch_types=[
        pltpu.VMEM((N,), jnp.float32),
        pltpu.VMEM((N,), jnp.float32),
        pltpu.VMEM((EPT,), jnp.int32),
        pltpu.VMEM((EPT,), jnp.int32),
        pltpu.VMEM((EPT,), jnp.float32),
        pltpu.VMEM((EPT,), jnp.float32),
    ],
    compiler_params=_SC_PARAMS_NL,
)


_NBLK = 1000
_NGRID = N // _NBLK
_EBLK = 8000
_EGRID = E // _EBLK


def _node_embed_tc(x_ref, w_ref, b_ref, out_ref):
    out_ref[...] = (
        jnp.dot(x_ref[...], w_ref[...], preferred_element_type=jnp.float32)
        + b_ref[...]
    )


def _pe_tc(e8_ref, wedge_ref, wpred_ref, bedge_ref, bpred_ref, out_ref):
    wpe = wpred_ref[...][2 * H:3 * H, :]
    wsmall = jnp.dot(wedge_ref[...], wpe, preferred_element_type=jnp.float32)
    c = jnp.dot(bedge_ref[...], wpe, preferred_element_type=jnp.float32)
    wrep = jnp.concatenate([wsmall] * 8, axis=0)
    row = lax.broadcasted_iota(jnp.int32, (8 * DE, 8), 0)
    col = lax.broadcasted_iota(jnp.int32, (8 * DE, 8), 1)
    w8 = jnp.where(row // DE == col, wrep, 0.0)
    out_ref[...] = (
        jnp.dot(e8_ref[...], w8, preferred_element_type=jnp.float32)
        + c + bpred_ref[...]
    )


def _norm_tc(dps_ref, dpd_ref, h0_ref, hs_ref, rso_ref, rsi_ref):
    ds_ = dps_ref[...]
    dd_ = dpd_ref[...]
    deg_s = ds_[0, :, 0:1] + ds_[1, :, 0:1]
    deg_d = dd_[0, :, 0:1] + dd_[1, :, 0:1]
    rso = lax.rsqrt(jnp.maximum(deg_s, 1.0))
    rsi = lax.rsqrt(jnp.maximum(deg_d, 1.0))
    hs_ref[...] = h0_ref[...] * rso
    rso_ref[...] = rso
    rsi_ref[...] = rsi


def _layer_tc(aggp_ref, rsi_ref, rso_ref, w_ref, b_ref, out_ref):
    a = aggp_ref[...]
    agg = (a[0] + a[1]) * rsi_ref[...]
    h = jnp.maximum(
        jnp.dot(agg, w_ref[...], preferred_element_type=jnp.float32) + b_ref[...],
        0.0,
    )
    out_ref[...] = h * rso_ref[...]


def _final_tc(aggp_ref, rsi_ref, w_ref, b_ref, wpred_ref, ps_ref, pd_ref):
    a = aggp_ref[...]
    agg = (a[0] + a[1]) * rsi_ref[...]
    h = jnp.maximum(
        jnp.dot(agg, w_ref[...], preferred_element_type=jnp.float32) + b_ref[...],
        0.0,
    )
    wp = wpred_ref[...]
    ps_ref[...] = jnp.dot(h, wp[0:H, :], preferred_element_type=jnp.float32)
    pd_ref[...] = jnp.dot(h, wp[H:2 * H, :], preferred_element_type=jnp.float32)


def _full(shape):
    return pl.BlockSpec(shape, lambda i: (0,) * len(shape))


_node_embed_call = pl.pallas_call(
    _node_embed_tc,
    grid=(_NGRID,),
    in_specs=[
        pl.BlockSpec((_NBLK, DN), lambda i: (i, 0)),
        _full((DN, H)),
        _full((1, H)),
    ],
    out_specs=pl.BlockSpec((_NBLK, H), lambda i: (i, 0)),
    out_shape=jax.ShapeDtypeStruct((N, H), jnp.float32),
)

_E8 = E // 8
_E8BLK = 4000

_pe_call = pl.pallas_call(
    _pe_tc,
    grid=(_E8 // _E8BLK,),
    in_specs=[
        pl.BlockSpec((_E8BLK, 8 * DE), lambda i: (i, 0)),
        _full((DE, H)),
        _full((3 * H, 1)),
        _full((1, H)),
        _full((1, 1)),
    ],
    out_specs=pl.BlockSpec((_E8BLK, 8), lambda i: (i, 0)),
    out_shape=jax.ShapeDtypeStruct((_E8, 8), jnp.float32),
)

_norm_call = pl.pallas_call(
    _norm_tc,
    grid=(_NGRID,),
    in_specs=[
        pl.BlockSpec((NC, _NBLK, L), lambda i: (0, i, 0)),
        pl.BlockSpec((NC, _NBLK, L), lambda i: (0, i + _NGRID, 0)),
        pl.BlockSpec((_NBLK, H), lambda i: (i, 0)),
    ],
    out_specs=[
        pl.BlockSpec((_NBLK, H), lambda i: (i, 0)),
        pl.BlockSpec((_NBLK, 1), lambda i: (i, 0)),
        pl.BlockSpec((_NBLK, 1), lambda i: (i, 0)),
    ],
    out_shape=[
        jax.ShapeDtypeStruct((N, H), jnp.float32),
        jax.ShapeDtypeStruct((N, 1), jnp.float32),
        jax.ShapeDtypeStruct((N, 1), jnp.float32),
    ],
)

_layer_call = pl.pallas_call(
    _layer_tc,
    grid=(_NGRID,),
    in_specs=[
        pl.BlockSpec((NC, _NBLK, H), lambda i: (0, i, 0)),
        pl.BlockSpec((_NBLK, 1), lambda i: (i, 0)),
        pl.BlockSpec((_NBLK, 1), lambda i: (i, 0)),
        _full((H, H)),
        _full((1, H)),
    ],
    out_specs=pl.BlockSpec((_NBLK, H), lambda i: (i, 0)),
    out_shape=jax.ShapeDtypeStruct((N, H), jnp.float32),
)

_final_call = pl.pallas_call(
    _final_tc,
    grid=(_NGRID,),
    in_specs=[
        pl.BlockSpec((NC, _NBLK, H), lambda i: (0, i, 0)),
        pl.BlockSpec((_NBLK, 1), lambda i: (i, 0)),
        _full((H, H)),
        _full((1, H)),
        _full((3 * H, 1)),
    ],
    out_specs=[
        pl.BlockSpec((_NBLK, 1), lambda i: (i, 0)),
        pl.BlockSpec((_NBLK, 1), lambda i: (i, 0)),
    ],
    out_shape=[
        jax.ShapeDtypeStruct((N, 1), jnp.float32),
        jax.ShapeDtypeStruct((N, 1), jnp.float32),
    ],
)


def kernel(x, e, edge_index, W_node, b_node, W_edge, b_edge, Wg0, bg0, Wg1, bg1,
           W_pred, b_pred):
    src = edge_index[0].astype(jnp.int32)
    dst = edge_index[1].astype(jnp.int32)
    idx2 = jnp.concatenate([src, dst + N])

    npad_d = KD * CH - 2 * E // NW
    pad_d = (2 * N + jnp.arange(NW * npad_d, dtype=jnp.int32) % (DNP - 2 * N)
             ).reshape(NW, npad_d)
    idx2p = jnp.concatenate(
        [idx2.reshape(NW, 2 * E // NW), pad_d], axis=1
    ).reshape(NW, KD, CH)

    h0 = _node_embed_call(x, W_node, b_node.reshape(1, H))
    pe = _pe_call(e.reshape(_E8, 8 * DE), W_edge, W_pred,
                  b_edge.reshape(1, H), b_pred.reshape(1, 1))
    degp = _deg_call(idx2p)

    hs1, rso, rsi = _norm_call(degp, degp, h0)
    aggp1 = _agg_call(hs1, src, dst)
    hs2 = _layer_call(aggp1, rsi, rso, Wg0, bg0.reshape(1, H))
    aggp2 = _agg_call(hs2, src, dst)
    ps, pd = _final_call(aggp2, rsi, Wg1, bg1.reshape(1, H), W_pred)

    scores = _score_call(ps.reshape(N), pd.reshape(N), src, dst, pe.reshape(E))
    return scores.reshape(E, 1)


# speedup vs baseline: 1.2696x; 1.0137x over previous
import functools

import jax
import jax.numpy as jnp
from jax import lax
from jax.experimental import pallas as pl
from jax.experimental.pallas import tpu as pltpu
from jax.experimental.pallas import tpu_sc as plsc

N = 10000
E = 320000
DN = 128
DE = 16
H = 128

NC = 2
NS = 16
L = 16
NW = NC * NS

_MESH = plsc.VectorSubcoreMesh(
    core_axis_name="c", subcore_axis_name="s", num_cores=NC, num_subcores=NS
)
_SC_PARAMS = pltpu.CompilerParams(use_tc_tiling_on_sc=False)
_SC_PARAMS_NL = pltpu.CompilerParams(
    use_tc_tiling_on_sc=False, needs_layout_passes=False
)

CH = 128
KD = 160
DEPTH = 4
NP = 10112
DNP = 20096
DROWS = DNP // NS
AROWS = NP // NS
ZR = 104
EPT = E // NW


def _worker_id():
    return lax.axis_index("s") * NC + lax.axis_index("c")


def _deg_body(idx2_hbm, out_hbm, deg_sh, ones_v, idxw_v, zrow_v, ssem):
    cid = lax.axis_index("c")
    sid = lax.axis_index("s")
    wid = _worker_id()

    @pl.loop(0, CH)
    def _(i):
        ones_v[i, :] = jnp.full((L,), 1.0, jnp.float32)

    @pl.loop(0, DROWS)
    def _(i):
        zrow_v[i, :] = jnp.zeros((L,), jnp.float32)

    pltpu.sync_copy(zrow_v, deg_sh.at[pl.ds(sid * DROWS, DROWS)])
    pltpu.sync_copy(idx2_hbm.at[wid], idxw_v)
    plsc.subcore_barrier()

    @pl.loop(0, KD // DEPTH)
    def _(j):
        for b in range(DEPTH):
            k = j * DEPTH + b

            @pl.when(j >= 1)
            def _():
                pltpu.make_async_copy(
                    ones_v, deg_sh.at[idxw_v.at[0]], ssem.at[b]
                ).wait()

            pltpu.async_copy(ones_v, deg_sh.at[idxw_v.at[k]], ssem.at[b], add=True)

    for b in range(DEPTH):
        pltpu.make_async_copy(ones_v, deg_sh.at[idxw_v.at[0]], ssem.at[b]).wait()

    plsc.subcore_barrier()
    pltpu.sync_copy(
        deg_sh.at[pl.ds(sid * DROWS, DROWS)],
        out_hbm.at[cid, pl.ds(sid * DROWS, DROWS)],
    )


_deg_call = pl.kernel(
    _deg_body,
    out_type=jax.ShapeDtypeStruct((NC, DNP, L), jnp.float32),
    mesh=_MESH,
    scratch_types=[
        pltpu.VMEM_SHARED((DNP, L), jnp.float32),
        pltpu.VMEM((CH, L), jnp.float32),
        pltpu.VMEM((KD, CH), jnp.int32),
        pltpu.VMEM((DROWS, L), jnp.float32),
        pltpu.SemaphoreType.DMA((DEPTH,)),
    ],
    compiler_params=_SC_PARAMS,
)


AGG_CHUNKS = E // CH
AGG_KMAX = -(-AGG_CHUNKS // NW)


def _agg_body(hs_hbm, src_hbm, dst_hbm, out_hbm, agg_sh, rows_v, sidx_v,
              didx_v, zbuf_v, gsem, ssem, isem):
    cid = lax.axis_index("c")
    sid = lax.axis_index("s")
    wid = _worker_id()

    @pl.loop(0, ZR)
    def _(i):
        @pl.loop(0, H // L)
        def _(j):
            zbuf_v[i, pl.ds(j * L, L)] = jnp.zeros((L,), jnp.float32)

    @pl.loop(0, AROWS // ZR)
    def _(r):
        pltpu.sync_copy(zbuf_v, agg_sh.at[pl.ds(sid * AROWS + r * ZR, ZR)])

    pltpu.sync_copy(
        zbuf_v.at[pl.ds(0, AROWS % ZR)],
        agg_sh.at[pl.ds(sid * AROWS + (AROWS // ZR) * ZR, AROWS % ZR)],
    )

    for k0 in (0, 1):
        c0 = wid + k0 * NW

        @pl.when(c0 < AGG_CHUNKS)
        def _():
            pltpu.async_copy(src_hbm.at[pl.ds(c0 * CH, CH)], sidx_v.at[k0],
                             isem.at[k0])
            pltpu.async_copy(dst_hbm.at[pl.ds(c0 * CH, CH)], didx_v.at[k0],
                             isem.at[k0])

    plsc.subcore_barrier()

    @pl.loop(0, (AGG_KMAX + 4) // 4)
    def _(j):
        for u in range(4):
            k = 4 * j + u
            c = wid + k * NW
            half = u % 2
            rows_h = rows_v.at[half]
            sidx_u = sidx_v.at[u]
            didx_u = didx_v.at[u]

            @pl.when(c < AGG_CHUNKS)
            def _():
                @pl.when(k >= 2)
                def _():
                    pltpu.make_async_copy(
                        rows_h, agg_sh.at[didx_u], ssem.at[half]
                    ).wait()

                @pl.when(c + 2 * NW < AGG_CHUNKS)
                def _():
                    c2 = c + 2 * NW
                    u2 = (u + 2) % 4
                    pltpu.async_copy(src_hbm.at[pl.ds(c2 * CH, CH)],
                                     sidx_v.at[u2], isem.at[u2])
                    pltpu.async_copy(dst_hbm.at[pl.ds(c2 * CH, CH)],
                                     didx_v.at[u2], isem.at[u2])

                pltpu.make_async_copy(src_hbm.at[pl.ds(c * CH, CH)], sidx_u,
                                      isem.at[u]).wait()
                pltpu.make_async_copy(dst_hbm.at[pl.ds(c * CH, CH)], didx_u,
                                      isem.at[u]).wait()
                pltpu.async_copy(hs_hbm.at[sidx_u], rows_h, gsem.at[half]).wait()
                pltpu.async_copy(rows_h, agg_sh.at[didx_u], ssem.at[half],
                                 add=True)

    for half in (0, 1):
        pltpu.make_async_copy(
            rows_v.at[half], agg_sh.at[didx_v.at[half]], ssem.at[half]
        ).wait()

    plsc.subcore_barrier()
    pltpu.sync_copy(
        agg_sh.at[pl.ds(sid * AROWS, AROWS)],
        out_hbm.at[cid, pl.ds(sid * AROWS, AROWS)],
    )


_agg_call = pl.kernel(
    _agg_body,
    out_type=jax.ShapeDtypeStruct((NC, NP, H), jnp.float32),
    mesh=_MESH,
    scratch_types=[
        pltpu.VMEM_SHARED((NP, H), jnp.float32),
        pltpu.VMEM((2, CH, H), jnp.float32),
        pltpu.VMEM((4, CH), jnp.int32),
        pltpu.VMEM((4, CH), jnp.int32),
        pltpu.VMEM((ZR, H), jnp.float32),
        pltpu.SemaphoreType.DMA((2,)),
        pltpu.SemaphoreType.DMA((2,)),
        pltpu.SemaphoreType.DMA((4,)),
    ],
    compiler_params=_SC_PARAMS,
)


def _score_body(ps_hbm, pd_hbm, src_hbm, dst_hbm, pe_hbm, out_hbm,
                ps_v, pd_v, sidx_v, didx_v, pe_v, out_v):
    wid = _worker_id()
    base = wid * EPT
    pltpu.sync_copy(ps_hbm, ps_v)
    pltpu.sync_copy(pd_hbm, pd_v)
    pltpu.sync_copy(src_hbm.at[pl.ds(base, EPT)], sidx_v)
    pltpu.sync_copy(dst_hbm.at[pl.ds(base, EPT)], didx_v)
    pltpu.sync_copy(pe_hbm.at[pl.ds(base, EPT)], pe_v)

    @pl.loop(0, EPT // L)
    def _(i):
        si = sidx_v[pl.ds(i * L, L)]
        di = didx_v[pl.ds(i * L, L)]
        vs = plsc.load_gather(ps_v, [si])
        vd = plsc.load_gather(pd_v, [di])
        out_v[pl.ds(i * L, L)] = vs + vd + pe_v[pl.ds(i * L, L)]

    pltpu.sync_copy(out_v, out_hbm.at[pl.ds(base, EPT)])


_score_call = pl.kernel(
    _score_body,
    out_type=jax.ShapeDtypeStruct((E,), jnp.float32),
    mesh=_MESH,
    scratch_types=[
        pltpu.VMEM((N,), jnp.float32),
        pltpu.VMEM((N,), jnp.float32),
        pltpu.VMEM((EPT,), jnp.int32),
        pltpu.VMEM((EPT,), jnp.int32),
        pltpu.VMEM((EPT,), jnp.float32),
        pltpu.VMEM((EPT,), jnp.float32),
    ],
    compiler_params=_SC_PARAMS_NL,
)


_NBLK = 1000
_NGRID = N // _NBLK


def _node_embed_tc(x_ref, w_ref, b_ref, out_ref):
    out_ref[...] = (
        jnp.dot(x_ref[...], w_ref[...], preferred_element_type=jnp.float32)
        + b_ref[...]
    )


def _pe_tc(e8_ref, wedge_ref, wpred_ref, bedge_ref, bpred_ref, out_ref):
    wpe = wpred_ref[...][2 * H:3 * H, :]
    wsmall = jnp.dot(wedge_ref[...], wpe, preferred_element_type=jnp.float32)
    c = jnp.dot(bedge_ref[...], wpe, preferred_element_type=jnp.float32)
    wrep = jnp.concatenate([wsmall] * 8, axis=0)
    row = lax.broadcasted_iota(jnp.int32, (8 * DE, 8), 0)
    col = lax.broadcasted_iota(jnp.int32, (8 * DE, 8), 1)
    w8 = jnp.where(row // DE == col, wrep, 0.0)
    out_ref[...] = (
        jnp.dot(e8_ref[...], w8, preferred_element_type=jnp.float32)
        + c + bpred_ref[...]
    )


def _norm_tc(dps_ref, dpd_ref, h0_ref, hs_ref, rso_ref, rsi_ref):
    ds_ = dps_ref[...]
    dd_ = dpd_ref[...]
    deg_s = ds_[0, :, 0:1] + ds_[1, :, 0:1]
    deg_d = dd_[0, :, 0:1] + dd_[1, :, 0:1]
    rso = lax.rsqrt(jnp.maximum(deg_s, 1.0))
    rsi = lax.rsqrt(jnp.maximum(deg_d, 1.0))
    hs_ref[...] = h0_ref[...] * rso
    rso_ref[...] = rso
    rsi_ref[...] = rsi


def _layer_tc(aggp_ref, rsi_ref, rso_ref, w_ref, b_ref, out_ref):
    a = aggp_ref[...]
    agg = (a[0] + a[1]) * rsi_ref[...]
    h = jnp.maximum(
        jnp.dot(agg, w_ref[...], preferred_element_type=jnp.float32) + b_ref[...],
        0.0,
    )
    out_ref[...] = h * rso_ref[...]


def _final_tc(aggp_ref, rsi_ref, w_ref, b_ref, wpred_ref, ps_ref, pd_ref):
    a = aggp_ref[...]
    agg = (a[0] + a[1]) * rsi_ref[...]
    h = jnp.maximum(
        jnp.dot(agg, w_ref[...], preferred_element_type=jnp.float32) + b_ref[...],
        0.0,
    )
    wp = wpred_ref[...]
    ps_ref[...] = jnp.dot(h, wp[0:H, :], preferred_element_type=jnp.float32)
    pd_ref[...] = jnp.dot(h, wp[H:2 * H, :], preferred_element_type=jnp.float32)


def _full(shape):
    return pl.BlockSpec(shape, lambda i: (0,) * len(shape))


_node_embed_call = pl.pallas_call(
    _node_embed_tc,
    grid=(_NGRID,),
    in_specs=[
        pl.BlockSpec((_NBLK, DN), lambda i: (i, 0)),
        _full((DN, H)),
        _full((1, H)),
    ],
    out_specs=pl.BlockSpec((_NBLK, H), lambda i: (i, 0)),
    out_shape=jax.ShapeDtypeStruct((N, H), jnp.float32),
)

_E8 = E // 8
_E8BLK = 4000

_pe_call = pl.pallas_call(
    _pe_tc,
    grid=(_E8 // _E8BLK,),
    in_specs=[
        pl.BlockSpec((_E8BLK, 8 * DE), lambda i: (i, 0)),
        _full((DE, H)),
        _full((3 * H, 1)),
        _full((1, H)),
        _full((1, 1)),
    ],
    out_specs=pl.BlockSpec((_E8BLK, 8), lambda i: (i, 0)),
    out_shape=jax.ShapeDtypeStruct((_E8, 8), jnp.float32),
)

_norm_call = pl.pallas_call(
    _norm_tc,
    grid=(_NGRID,),
    in_specs=[
        pl.BlockSpec((NC, _NBLK, L), lambda i: (0, i, 0)),
        pl.BlockSpec((NC, _NBLK, L), lambda i: (0, i + _NGRID, 0)),
        pl.BlockSpec((_NBLK, H), lambda i: (i, 0)),
    ],
    out_specs=[
        pl.BlockSpec((_NBLK, H), lambda i: (i, 0)),
        pl.BlockSpec((_NBLK, 1), lambda i: (i, 0)),
        pl.BlockSpec((_NBLK, 1), lambda i: (i, 0)),
    ],
    out_shape=[
        jax.ShapeDtypeStruct((N, H), jnp.float32),
        jax.ShapeDtypeStruct((N, 1), jnp.float32),
        jax.ShapeDtypeStruct((N, 1), jnp.float32),
    ],
)

_layer_call = pl.pallas_call(
    _layer_tc,
    grid=(_NGRID,),
    in_specs=[
        pl.BlockSpec((NC, _NBLK, H), lambda i: (0, i, 0)),
        pl.BlockSpec((_NBLK, 1), lambda i: (i, 0)),
        pl.BlockSpec((_NBLK, 1), lambda i: (i, 0)),
        _full((H, H)),
        _full((1, H)),
    ],
    out_specs=pl.BlockSpec((_NBLK, H), lambda i: (i, 0)),
    out_shape=jax.ShapeDtypeStruct((N, H), jnp.float32),
)

_final_call = pl.pallas_call(
    _final_tc,
    grid=(_NGRID,),
    in_specs=[
        pl.BlockSpec((NC, _NBLK, H), lambda i: (0, i, 0)),
        pl.BlockSpec((_NBLK, 1), lambda i: (i, 0)),
        _full((H, H)),
        _full((1, H)),
        _full((3 * H, 1)),
    ],
    out_specs=[
        pl.BlockSpec((_NBLK, 1), lambda i: (i, 0)),
        pl.BlockSpec((_NBLK, 1), lambda i: (i, 0)),
    ],
    out_shape=[
        jax.ShapeDtypeStruct((N, 1), jnp.float32),
        jax.ShapeDtypeStruct((N, 1), jnp.float32),
    ],
)


def kernel(x, e, edge_index, W_node, b_node, W_edge, b_edge, Wg0, bg0, Wg1, bg1,
           W_pred, b_pred):
    src = edge_index[0].astype(jnp.int32)
    dst = edge_index[1].astype(jnp.int32)
    idx2 = jnp.concatenate([src, dst + N])

    npad_d = KD * CH - 2 * E // NW
    pad_d = (2 * N + jnp.arange(NW * npad_d, dtype=jnp.int32) % (DNP - 2 * N)
             ).reshape(NW, npad_d)
    idx2p = jnp.concatenate(
        [idx2.reshape(NW, 2 * E // NW), pad_d], axis=1
    ).reshape(NW, KD, CH)

    h0 = _node_embed_call(x, W_node, b_node.reshape(1, H))
    pe = _pe_call(e.reshape(_E8, 8 * DE), W_edge, W_pred,
                  b_edge.reshape(1, H), b_pred.reshape(1, 1))
    degp = _deg_call(idx2p)

    hs1, rso, rsi = _norm_call(degp, degp, h0)
    aggp1 = _agg_call(hs1, src, dst)
    hs2 = _layer_call(aggp1, rsi, rso, Wg0, bg0.reshape(1, H))
    aggp2 = _agg_call(hs2, src, dst)
    ps, pd = _final_call(aggp2, rsi, Wg1, bg1.reshape(1, H), W_pred)

    scores = _score_call(ps.reshape(N), pd.reshape(N), src, dst, pe.reshape(E))
    return scores.reshape(E, 1)


# speedup vs baseline: 1.3368x; 1.0529x over previous
import functools

import jax
import jax.numpy as jnp
from jax import lax
from jax.experimental import pallas as pl
from jax.experimental.pallas import tpu as pltpu
from jax.experimental.pallas import tpu_sc as plsc

N = 10000
E = 320000
DN = 128
DE = 16
H = 128

NC = 2
NS = 16
L = 16
NW = NC * NS

_MESH = plsc.VectorSubcoreMesh(
    core_axis_name="c", subcore_axis_name="s", num_cores=NC, num_subcores=NS
)
_SC_PARAMS = pltpu.CompilerParams(use_tc_tiling_on_sc=False)
_SC_PARAMS_NL = pltpu.CompilerParams(
    use_tc_tiling_on_sc=False, needs_layout_passes=False
)

CH = 128
KD = 160
DEPTH = 4
NP = 10112
DNP = 20096
DROWS = DNP // NS
AROWS = NP // NS
ZR = 104
EPT = E // NW


def _worker_id():
    return lax.axis_index("s") * NC + lax.axis_index("c")


def _deg_body(idx2_hbm, out_hbm, deg_sh, ones_v, idxw_v, zrow_v, ssem):
    cid = lax.axis_index("c")
    sid = lax.axis_index("s")
    wid = _worker_id()

    @pl.loop(0, CH)
    def _(i):
        ones_v[i, :] = jnp.full((L,), 1.0, jnp.float32)

    @pl.loop(0, DROWS)
    def _(i):
        zrow_v[i, :] = jnp.zeros((L,), jnp.float32)

    pltpu.sync_copy(zrow_v, deg_sh.at[pl.ds(sid * DROWS, DROWS)])
    pltpu.sync_copy(idx2_hbm.at[wid], idxw_v)
    plsc.subcore_barrier()

    @pl.loop(0, KD // DEPTH)
    def _(j):
        for b in range(DEPTH):
            k = j * DEPTH + b

            @pl.when(j >= 1)
            def _():
                pltpu.make_async_copy(
                    ones_v, deg_sh.at[idxw_v.at[0]], ssem.at[b]
                ).wait()

            pltpu.async_copy(ones_v, deg_sh.at[idxw_v.at[k]], ssem.at[b], add=True)

    for b in range(DEPTH):
        pltpu.make_async_copy(ones_v, deg_sh.at[idxw_v.at[0]], ssem.at[b]).wait()

    plsc.subcore_barrier()
    pltpu.sync_copy(
        deg_sh.at[pl.ds(sid * DROWS, DROWS)],
        out_hbm.at[cid, pl.ds(sid * DROWS, DROWS)],
    )


_deg_call = pl.kernel(
    _deg_body,
    out_type=jax.ShapeDtypeStruct((NC, DNP, L), jnp.float32),
    mesh=_MESH,
    scratch_types=[
        pltpu.VMEM_SHARED((DNP, L), jnp.float32),
        pltpu.VMEM((CH, L), jnp.float32),
        pltpu.VMEM((KD, CH), jnp.int32),
        pltpu.VMEM((DROWS, L), jnp.float32),
        pltpu.SemaphoreType.DMA((DEPTH,)),
    ],
    compiler_params=_SC_PARAMS,
)


AGG_CHUNKS = E // CH
AGG_KMAX = -(-AGG_CHUNKS // NW)


def _agg_body(hs_hbm, src_hbm, dst_hbm, out_hbm, agg_sh, rows_v, sidx_v,
              didx_v, zbuf_v, gsem, ssem, isem):
    cid = lax.axis_index("c")
    sid = lax.axis_index("s")
    wid = _worker_id()

    @pl.loop(0, ZR)
    def _(i):
        @pl.loop(0, H // L)
        def _(j):
            zbuf_v[i, pl.ds(j * L, L)] = jnp.zeros((L,), jnp.float32)

    @pl.loop(0, AROWS // ZR)
    def _(r):
        pltpu.sync_copy(zbuf_v, agg_sh.at[pl.ds(sid * AROWS + r * ZR, ZR)])

    pltpu.sync_copy(
        zbuf_v.at[pl.ds(0, AROWS % ZR)],
        agg_sh.at[pl.ds(sid * AROWS + (AROWS // ZR) * ZR, AROWS % ZR)],
    )

    for k0 in (0, 1):
        c0 = wid + k0 * NW

        @pl.when(c0 < AGG_CHUNKS)
        def _():
            pltpu.async_copy(src_hbm.at[pl.ds(c0 * CH, CH)], sidx_v.at[k0],
                             isem.at[k0])
            pltpu.async_copy(dst_hbm.at[pl.ds(c0 * CH, CH)], didx_v.at[k0],
                             isem.at[k0])

    plsc.subcore_barrier()

    @pl.loop(0, (AGG_KMAX + 4) // 4)
    def _(j):
        for u in range(4):
            k = 4 * j + u
            c = wid + k * NW
            half = u % 2
            rows_h = rows_v.at[half]
            sidx_u = sidx_v.at[u]
            didx_u = didx_v.at[u]

            @pl.when(c < AGG_CHUNKS)
            def _():
                @pl.when(k >= 2)
                def _():
                    pltpu.make_async_copy(
                        rows_h, agg_sh.at[didx_u], ssem.at[half]
                    ).wait()

                pltpu.make_async_copy(src_hbm.at[pl.ds(c * CH, CH)], sidx_u,
                                      isem.at[u]).wait()
                pltpu.make_async_copy(dst_hbm.at[pl.ds(c * CH, CH)], didx_u,
                                      isem.at[u]).wait()
                pltpu.async_copy(hs_hbm.at[sidx_u], rows_h, gsem.at[half])

                @pl.when(c + 2 * NW < AGG_CHUNKS)
                def _():
                    c2 = c + 2 * NW
                    u2 = (u + 2) % 4
                    pltpu.async_copy(src_hbm.at[pl.ds(c2 * CH, CH)],
                                     sidx_v.at[u2], isem.at[u2])
                    pltpu.async_copy(dst_hbm.at[pl.ds(c2 * CH, CH)],
                                     didx_v.at[u2], isem.at[u2])

            @pl.when((k >= 1) & (c - NW < AGG_CHUNKS))
            def _():
                km = (k - 1) % 4
                rows_m = rows_v.at[1 - half]
                pltpu.make_async_copy(
                    hs_hbm.at[sidx_v.at[km]], rows_m, gsem.at[1 - half]
                ).wait()
                pltpu.async_copy(rows_m, agg_sh.at[didx_v.at[km]],
                                 ssem.at[1 - half], add=True)

    for half in (0, 1):
        pltpu.make_async_copy(
            rows_v.at[half], agg_sh.at[didx_v.at[half]], ssem.at[half]
        ).wait()

    plsc.subcore_barrier()
    pltpu.sync_copy(
        agg_sh.at[pl.ds(sid * AROWS, AROWS)],
        out_hbm.at[cid, pl.ds(sid * AROWS, AROWS)],
    )


_agg_call = pl.kernel(
    _agg_body,
    out_type=jax.ShapeDtypeStruct((NC, NP, H), jnp.float32),
    mesh=_MESH,
    scratch_types=[
        pltpu.VMEM_SHARED((NP, H), jnp.float32),
        pltpu.VMEM((2, CH, H), jnp.float32),
        pltpu.VMEM((4, CH), jnp.int32),
        pltpu.VMEM((4, CH), jnp.int32),
        pltpu.VMEM((ZR, H), jnp.float32),
        pltpu.SemaphoreType.DMA((2,)),
        pltpu.SemaphoreType.DMA((2,)),
        pltpu.SemaphoreType.DMA((4,)),
    ],
    compiler_params=_SC_PARAMS,
)


def _score_body(ps_hbm, pd_hbm, src_hbm, dst_hbm, pe_hbm, out_hbm,
                ps_v, pd_v, sidx_v, didx_v, pe_v, out_v):
    wid = _worker_id()
    base = wid * EPT
    pltpu.sync_copy(ps_hbm, ps_v)
    pltpu.sync_copy(pd_hbm, pd_v)
    pltpu.sync_copy(src_hbm.at[pl.ds(base, EPT)], sidx_v)
    pltpu.sync_copy(dst_hbm.at[pl.ds(base, EPT)], didx_v)
    pltpu.sync_copy(pe_hbm.at[pl.ds(base, EPT)], pe_v)

    @pl.loop(0, EPT // L)
    def _(i):
        si = sidx_v[pl.ds(i * L, L)]
        di = didx_v[pl.ds(i * L, L)]
        vs = plsc.load_gather(ps_v, [si])
        vd = plsc.load_gather(pd_v, [di])
        out_v[pl.ds(i * L, L)] = vs + vd + pe_v[pl.ds(i * L, L)]

    pltpu.sync_copy(out_v, out_hbm.at[pl.ds(base, EPT)])


_score_call = pl.kernel(
    _score_body,
    out_type=jax.ShapeDtypeStruct((E,), jnp.float32),
    mesh=_MESH,
    scratch_types=[
        pltpu.VMEM((N,), jnp.float32),
        pltpu.VMEM((N,), jnp.float32),
        pltpu.VMEM((EPT,), jnp.int32),
        pltpu.VMEM((EPT,), jnp.int32),
        pltpu.VMEM((EPT,), jnp.float32),
        pltpu.VMEM((EPT,), jnp.float32),
    ],
    compiler_params=_SC_PARAMS_NL,
)


_NBLK = 1000
_NGRID = N // _NBLK


def _node_embed_tc(x_ref, w_ref, b_ref, out_ref):
    out_ref[...] = (
        jnp.dot(x_ref[...], w_ref[...], preferred_element_type=jnp.float32)
        + b_ref[...]
    )


def _pe_tc(e8_ref, wedge_ref, wpred_ref, bedge_ref, bpred_ref, out_ref):
    wpe = wpred_ref[...][2 * H:3 * H, :]
    wsmall = jnp.dot(wedge_ref[...], wpe, preferred_element_type=jnp.float32)
    c = jnp.dot(bedge_ref[...], wpe, preferred_element_type=jnp.float32)
    wrep = jnp.concatenate([wsmall] * 8, axis=0)
    row = lax.broadcasted_iota(jnp.int32, (8 * DE, 8), 0)
    col = lax.broadcasted_iota(jnp.int32, (8 * DE, 8), 1)
    w8 = jnp.where(row // DE == col, wrep, 0.0)
    out_ref[...] = (
        jnp.dot(e8_ref[...], w8, preferred_element_type=jnp.float32)
        + c + bpred_ref[...]
    )


def _norm_tc(dps_ref, dpd_ref, h0_ref, hs_ref, rso_ref, rsi_ref):
    ds_ = dps_ref[...]
    dd_ = dpd_ref[...]
    deg_s = ds_[0, :, 0:1] + ds_[1, :, 0:1]
    deg_d = dd_[0, :, 0:1] + dd_[1, :, 0:1]
    rso = lax.rsqrt(jnp.maximum(deg_s, 1.0))
    rsi = lax.rsqrt(jnp.maximum(deg_d, 1.0))
    hs_ref[...] = h0_ref[...] * rso
    rso_ref[...] = rso
    rsi_ref[...] = rsi


def _layer_tc(aggp_ref, rsi_ref, rso_ref, w_ref, b_ref, out_ref):
    a = aggp_ref[...]
    agg = (a[0] + a[1]) * rsi_ref[...]
    h = jnp.maximum(
        jnp.dot(agg, w_ref[...], preferred_element_type=jnp.float32) + b_ref[...],
        0.0,
    )
    out_ref[...] = h * rso_ref[...]


def _final_tc(aggp_ref, rsi_ref, w_ref, b_ref, wpred_ref, ps_ref, pd_ref):
    a = aggp_ref[...]
    agg = (a[0] + a[1]) * rsi_ref[...]
    h = jnp.maximum(
        jnp.dot(agg, w_ref[...], preferred_element_type=jnp.float32) + b_ref[...],
        0.0,
    )
    wp = wpred_ref[...]
    ps_ref[...] = jnp.dot(h, wp[0:H, :], preferred_element_type=jnp.float32)
    pd_ref[...] = jnp.dot(h, wp[H:2 * H, :], preferred_element_type=jnp.float32)


def _full(shape):
    return pl.BlockSpec(shape, lambda i: (0,) * len(shape))


_node_embed_call = pl.pallas_call(
    _node_embed_tc,
    grid=(_NGRID,),
    in_specs=[
        pl.BlockSpec((_NBLK, DN), lambda i: (i, 0)),
        _full((DN, H)),
        _full((1, H)),
    ],
    out_specs=pl.BlockSpec((_NBLK, H), lambda i: (i, 0)),
    out_shape=jax.ShapeDtypeStruct((N, H), jnp.float32),
)

_E8 = E // 8
_E8BLK = 4000

_pe_call = pl.pallas_call(
    _pe_tc,
    grid=(_E8 // _E8BLK,),
    in_specs=[
        pl.BlockSpec((_E8BLK, 8 * DE), lambda i: (i, 0)),
        _full((DE, H)),
        _full((3 * H, 1)),
        _full((1, H)),
        _full((1, 1)),
    ],
    out_specs=pl.BlockSpec((_E8BLK, 8), lambda i: (i, 0)),
    out_shape=jax.ShapeDtypeStruct((_E8, 8), jnp.float32),
)

_norm_call = pl.pallas_call(
    _norm_tc,
    grid=(_NGRID,),
    in_specs=[
        pl.BlockSpec((NC, _NBLK, L), lambda i: (0, i, 0)),
        pl.BlockSpec((NC, _NBLK, L), lambda i: (0, i + _NGRID, 0)),
        pl.BlockSpec((_NBLK, H), lambda i: (i, 0)),
    ],
    out_specs=[
        pl.BlockSpec((_NBLK, H), lambda i: (i, 0)),
        pl.BlockSpec((_NBLK, 1), lambda i: (i, 0)),
        pl.BlockSpec((_NBLK, 1), lambda i: (i, 0)),
    ],
    out_shape=[
        jax.ShapeDtypeStruct((N, H), jnp.float32),
        jax.ShapeDtypeStruct((N, 1), jnp.float32),
        jax.ShapeDtypeStruct((N, 1), jnp.float32),
    ],
)

_layer_call = pl.pallas_call(
    _layer_tc,
    grid=(_NGRID,),
    in_specs=[
        pl.BlockSpec((NC, _NBLK, H), lambda i: (0, i, 0)),
        pl.BlockSpec((_NBLK, 1), lambda i: (i, 0)),
        pl.BlockSpec((_NBLK, 1), lambda i: (i, 0)),
        _full((H, H)),
        _full((1, H)),
    ],
    out_specs=pl.BlockSpec((_NBLK, H), lambda i: (i, 0)),
    out_shape=jax.ShapeDtypeStruct((N, H), jnp.float32),
)

_final_call = pl.pallas_call(
    _final_tc,
    grid=(_NGRID,),
    in_specs=[
        pl.BlockSpec((NC, _NBLK, H), lambda i: (0, i, 0)),
        pl.BlockSpec((_NBLK, 1), lambda i: (i, 0)),
        _full((H, H)),
        _full((1, H)),
        _full((3 * H, 1)),
    ],
    out_specs=[
        pl.BlockSpec((_NBLK, 1), lambda i: (i, 0)),
        pl.BlockSpec((_NBLK, 1), lambda i: (i, 0)),
    ],
    out_shape=[
        jax.ShapeDtypeStruct((N, 1), jnp.float32),
        jax.ShapeDtypeStruct((N, 1), jnp.float32),
    ],
)


def kernel(x, e, edge_index, W_node, b_node, W_edge, b_edge, Wg0, bg0, Wg1, bg1,
           W_pred, b_pred):
    src = edge_index[0].astype(jnp.int32)
    dst = edge_index[1].astype(jnp.int32)
    idx2 = jnp.concatenate([src, dst + N])

    npad_d = KD * CH - 2 * E // NW
    pad_d = (2 * N + jnp.arange(NW * npad_d, dtype=jnp.int32) % (DNP - 2 * N)
             ).reshape(NW, npad_d)
    idx2p = jnp.concatenate(
        [idx2.reshape(NW, 2 * E // NW), pad_d], axis=1
    ).reshape(NW, KD, CH)

    h0 = _node_embed_call(x, W_node, b_node.reshape(1, H))
    pe = _pe_call(e.reshape(_E8, 8 * DE), W_edge, W_pred,
                  b_edge.reshape(1, H), b_pred.reshape(1, 1))
    degp = _deg_call(idx2p)

    hs1, rso, rsi = _norm_call(degp, degp, h0)
    aggp1 = _agg_call(hs1, src, dst)
    hs2 = _layer_call(aggp1, rsi, rso, Wg0, bg0.reshape(1, H))
    aggp2 = _agg_call(hs2, src, dst)
    ps, pd = _final_call(aggp2, rsi, Wg1, bg1.reshape(1, H), W_pred)

    scores = _score_call(ps.reshape(N), pd.reshape(N), src, dst, pe.reshape(E))
    return scores.reshape(E, 1)
